# Initial kernel scaffold; baseline (speedup 1.0000x reference)
import jax
import jax.numpy as jnp
from jax.experimental import pallas as pl


def kernel(z, rbf, sbf, i, j, idx_kj, idx_ji, batch, params):
    raise NotImplementedError("write your pallas kernel here")


# speedup vs baseline: 6.1256x; 6.1256x over previous
import functools

import jax
import jax.numpy as jnp
from jax import lax
from jax.experimental import pallas as pl
from jax.experimental.pallas import tpu as pltpu
from jax.experimental.pallas import tpu_sc as plsc

_N = 10000
_E = 160000
_T = 640000
_NG = 64
_HC = 128
_IE = 64
_NR = 6
_SBF = 42
_OE = 256
_NL = 4

_NC = 2
_NS = 16

_BLK_E = 640
_BLK_T = 1024
_BLK_N = 400
_CHUNK_E = 10240
_NCHUNK = 16
_CT = 512
_CG = 200

_f32 = jnp.float32
_i32 = jnp.int32


def _swish(x):
    return x * (1.0 / (1.0 + jnp.exp(-x)))


def _full_spec(a):
    nd = a.ndim
    return pl.BlockSpec(a.shape, lambda b, _nd=nd: (0,) * _nd)


def _row_spec(blk, d):
    return pl.BlockSpec((blk, d), lambda b: (b, 0))


def _edense(body, grid, data_specs, weights, out_shapes, out_specs, *data):
    in_specs = list(data_specs) + [_full_spec(w) for w in weights]
    return pl.pallas_call(
        body,
        grid=grid,
        in_specs=in_specs,
        out_specs=out_specs,
        out_shape=out_shapes,
    )(*data, *weights)


def _k_emb(z2d, table):
    def body(z_ref, t_ref, o_ref):
        z = z_ref[...]
        oh = (z == lax.broadcasted_iota(_i32, (1, 128), 1)).astype(_f32)
        o_ref[...] = jnp.dot(oh, t_ref[...], preferred_element_type=_f32)

    return _edense(
        body, (_N // _BLK_N,), [_row_spec(_BLK_N, 1)], [table],
        jax.ShapeDtypeStruct((_N, _HC), _f32), _row_spec(_BLK_N, _HC), z2d)


def _k_init(xi, xj, rbf, W0, b0, Wa, Wb, Wc, bc, W1):
    def body(xi_ref, xj_ref, rbf_ref, W0r, b0r, War, Wbr, Wcr, bcr, W1r,
             e1_ref, e2_ref):
        rbf_v = rbf_ref[...]
        rbf0 = _swish(jnp.dot(rbf_v, W0r[...], preferred_element_type=_f32)
                      + b0r[...])
        e1 = _swish(jnp.dot(xi_ref[...], War[...], preferred_element_type=_f32)
                    + jnp.dot(xj_ref[...], Wbr[...], preferred_element_type=_f32)
                    + jnp.dot(rbf0, Wcr[...], preferred_element_type=_f32)
                    + bcr[...])
        e1_ref[...] = e1
        e2_ref[...] = jnp.dot(rbf_v, W1r[...], preferred_element_type=_f32) * e1

    return _edense(
        body, (_E // _BLK_E,),
        [_row_spec(_BLK_E, _HC), _row_spec(_BLK_E, _HC), _row_spec(_BLK_E, _NR)],
        [W0, b0, Wa, Wb, Wc, bc, W1],
        [jax.ShapeDtypeStruct((_E, _HC), _f32)] * 2,
        [_row_spec(_BLK_E, _HC)] * 2,
        xi, xj, rbf)


def _k_pre(e1, rbf, Wji, bji, Wkj, bkj, Wr, Wd):
    def body(x_ref, rbf_ref, Wjir, bjir, Wkjr, bkjr, Wrr, Wdr,
             xji_ref, xd_ref):
        x1 = x_ref[...]
        xji_ref[...] = _swish(
            jnp.dot(x1, Wjir[...], preferred_element_type=_f32) + bjir[...])
        xkj = _swish(
            jnp.dot(x1, Wkjr[...], preferred_element_type=_f32) + bkjr[...])
        r = jnp.dot(rbf_ref[...], Wrr[...], preferred_element_type=_f32)
        xd_ref[...] = _swish(
            jnp.dot(xkj * r, Wdr[...], preferred_element_type=_f32))

    return _edense(
        body, (_E // _BLK_E,),
        [_row_spec(_BLK_E, _HC), _row_spec(_BLK_E, _NR)],
        [Wji, bji, Wkj, bkj, Wr, Wd],
        [jax.ShapeDtypeStruct((_E, _HC), _f32),
         jax.ShapeDtypeStruct((_E, _IE), _f32)],
        [_row_spec(_BLK_E, _HC), _row_spec(_BLK_E, _IE)],
        e1, rbf)


def _k_s(sbf, Ws):
    def body(s_ref, Wr, o_ref):
        o_ref[...] = jnp.dot(s_ref[...], Wr[...], preferred_element_type=_f32)

    return _edense(
        body, (_T // _BLK_T,), [_row_spec(_BLK_T, _SBF)], [Ws],
        jax.ShapeDtypeStruct((_T, _IE), _f32), _row_spec(_BLK_T, _IE), sbf)


def _k_post(agg, xji, x1, rbf, Wup, Wb1, bb1, Wb2, bb2, Wl, bl,
            Wa11, ba11, Wa12, ba12, Wa21, ba21, Wa22, ba22, Wrbf):
    def body(agg_ref, xji_ref, x1_ref, rbf_ref, Wupr, Wb1r, bb1r, Wb2r, bb2r,
             Wlr, blr, Wa11r, ba11r, Wa12r, ba12r, Wa21r, ba21r, Wa22r, ba22r,
             Wrbfr, e1_ref, e2_ref):
        def linr(v, w, b):
            return jnp.dot(v, w[...], preferred_element_type=_f32) + b[...]

        xup = _swish(jnp.dot(agg_ref[...], Wupr[...],
                             preferred_element_type=_f32))
        e1 = xji_ref[...] + xup
        e1 = e1 + _swish(linr(_swish(linr(e1, Wb1r, bb1r)), Wb2r, bb2r))
        e1 = _swish(linr(e1, Wlr, blr)) + x1_ref[...]
        e1 = e1 + _swish(linr(_swish(linr(e1, Wa11r, ba11r)), Wa12r, ba12r))
        e1 = e1 + _swish(linr(_swish(linr(e1, Wa21r, ba21r)), Wa22r, ba22r))
        e1_ref[...] = e1
        e2_ref[...] = jnp.dot(rbf_ref[...], Wrbfr[...],
                              preferred_element_type=_f32) * e1

    return _edense(
        body, (_E // _BLK_E,),
        [_row_spec(_BLK_E, _IE), _row_spec(_BLK_E, _HC),
         _row_spec(_BLK_E, _HC), _row_spec(_BLK_E, _NR)],
        [Wup, Wb1, bb1, Wb2, bb2, Wl, bl, Wa11, ba11, Wa12, ba12,
         Wa21, ba21, Wa22, ba22, Wrbf],
        [jax.ShapeDtypeStruct((_E, _HC), _f32)] * 2,
        [_row_spec(_BLK_E, _HC)] * 2,
        agg, xji, x1, rbf)


def _k_out(va, vb, batch2d, Wup, bup, W1, b1, W2, b2, W3, b3, Wf):
    def body(va_ref, vb_ref, bt_ref, Wupr, bupr, W1r, b1r, W2r, b2r, W3r, b3r,
             Wfr, u_ref):
        v = va_ref[...] + vb_ref[...]
        v = jnp.dot(v, Wupr[...], preferred_element_type=_f32) + bupr[...]
        v = _swish(jnp.dot(v, W1r[...], preferred_element_type=_f32) + b1r[...])
        v = _swish(jnp.dot(v, W2r[...], preferred_element_type=_f32) + b2r[...])
        v = _swish(jnp.dot(v, W3r[...], preferred_element_type=_f32) + b3r[...])
        vo = jnp.dot(v, Wfr[...], preferred_element_type=_f32)
        oh = (bt_ref[...] == lax.broadcasted_iota(_i32, (1, _NG), 1)
              ).astype(_f32)
        u = lax.dot_general(oh, vo, (((0,), (0,)), ((), ())),
                            preferred_element_type=_f32)

        @pl.when(pl.program_id(0) == 0)
        def _():
            u_ref[...] = jnp.zeros_like(u_ref)

        u_ref[...] += u

    return _edense(
        body, (_N // _BLK_N,),
        [_row_spec(_BLK_N, _HC), _row_spec(_BLK_N, _HC), _row_spec(_BLK_N, 1)],
        [Wup, bup, W1, b1, W2, b2, W3, b3, Wf],
        jax.ShapeDtypeStruct((_NG, 1), _f32),
        pl.BlockSpec((_NG, 1), lambda b: (0, 0)),
        va, vb, batch2d)


def _sc_mesh():
    return plsc.VectorSubcoreMesh(core_axis_name="c", subcore_axis_name="s",
                                  num_cores=_NC, num_subcores=_NS)


def _zero_vmem(ref, rows, cols):
    z16 = jnp.zeros((16,), _f32)

    def body(r, carry):
        for cc in range(cols // 16):
            ref[r, pl.ds(cc * 16, 16)] = z16
        return carry

    lax.fori_loop(0, rows, body, None)


def _sc_gather_xij(x, iarr, jarr):
    per_tec = _E // (_NC * _NS)

    @functools.partial(
        pl.kernel,
        out_type=[jax.ShapeDtypeStruct((_E, _HC), _f32)] * 2,
        mesh=_sc_mesh(),
        compiler_params=pltpu.CompilerParams(needs_layout_passes=False, use_tc_tiling_on_sc=False),
        scratch_types=[
            pltpu.MemorySpace.VMEM((_CG,), _i32),
            pltpu.MemorySpace.VMEM((_CG, _HC), _f32),
            pltpu.SemaphoreType.DMA,
        ],
    )
    def k(x_hbm, i_hbm, j_hbm, xi_hbm, xj_hbm, idx_v, row_v, sem):
        c = lax.axis_index("c")
        s = lax.axis_index("s")
        base = (c * _NS + s) * per_tec

        def body(kk, carry):
            t = pl.multiple_of(base + kk * _CG, 8)
            pltpu.sync_copy(i_hbm.at[pl.ds(t, _CG)], idx_v)
            pltpu.async_copy(x_hbm.at[idx_v], row_v, sem).wait()
            pltpu.sync_copy(row_v, xi_hbm.at[pl.ds(t, _CG)])
            pltpu.sync_copy(j_hbm.at[pl.ds(t, _CG)], idx_v)
            pltpu.async_copy(x_hbm.at[idx_v], row_v, sem).wait()
            pltpu.sync_copy(row_v, xj_hbm.at[pl.ds(t, _CG)])
            return carry

        lax.fori_loop(0, per_tec // _CG, body, None)

    return k(x, iarr, jarr)


def _sc_segsum_v(e2, iarr):
    per_tec = _E // (_NC * _NS)
    zrows = 80

    @functools.partial(
        pl.kernel,
        out_type=jax.ShapeDtypeStruct((_NC * _N, _HC), _f32),
        mesh=_sc_mesh(),
        compiler_params=pltpu.CompilerParams(needs_layout_passes=False, use_tc_tiling_on_sc=False),
        scratch_types=[
            pltpu.MemorySpace.VMEM_SHARED((_N, _HC), _f32),
            pltpu.MemorySpace.VMEM((_CG,), _i32),
            pltpu.MemorySpace.VMEM((_CG, _HC), _f32),
            pltpu.MemorySpace.VMEM((zrows, _HC), _f32),
        ],
    )
    def k(e2_hbm, i_hbm, out_hbm, acc, idx_v, row_v, zbuf):
        c = lax.axis_index("c")
        s = lax.axis_index("s")
        _zero_vmem(zbuf, zrows, _HC)
        nz = jnp.where(s == 15, 400 // zrows, 640 // zrows)

        def zb(q, carry):
            off = pl.multiple_of(s * 640 + q * zrows, 8)
            pltpu.sync_copy(zbuf, acc.at[pl.ds(off, zrows)])
            return carry

        lax.fori_loop(0, nz, zb, None)
        plsc.subcore_barrier()

        base = c * (_E // _NC) + s * per_tec

        def body(kk, carry):
            t = pl.multiple_of(base + kk * _CG, 8)
            pltpu.sync_copy(i_hbm.at[pl.ds(t, _CG)], idx_v)
            pltpu.sync_copy(e2_hbm.at[pl.ds(t, _CG)], row_v)
            pltpu.sync_copy(row_v, acc.at[idx_v], add=True)
            return carry

        lax.fori_loop(0, per_tec // _CG, body, None)
        plsc.subcore_barrier()

        def wr(q, carry):
            off = pl.multiple_of(s * 640 + q * zrows, 8)
            pltpu.sync_copy(acc.at[pl.ds(off, zrows)],
                            out_hbm.at[pl.ds(pl.multiple_of(
                                c * _N + off, 8), zrows)])
            return carry

        lax.fori_loop(0, nz, wr, None)

    return k(e2, iarr)


def _sc_triplet(xd, s_tab, skj_p, sji_p, perm_p, bounds):
    @functools.partial(
        pl.kernel,
        out_type=jax.ShapeDtypeStruct((_E, _IE), _f32),
        mesh=_sc_mesh(),
        compiler_params=pltpu.CompilerParams(needs_layout_passes=False, use_tc_tiling_on_sc=False),
        scratch_types=[
            pltpu.MemorySpace.VMEM_SHARED((_CHUNK_E + 16, _IE), _f32),
            pltpu.MemorySpace.VMEM((_CT,), _i32),
            pltpu.MemorySpace.VMEM((_CT,), _i32),
            pltpu.MemorySpace.VMEM((_CT,), _i32),
            pltpu.MemorySpace.VMEM((_CT,), _i32),
            pltpu.MemorySpace.VMEM((32,), _i32),
            pltpu.MemorySpace.VMEM((_CT, _IE), _f32),
            pltpu.MemorySpace.VMEM((_CT, _IE), _f32),
            pltpu.SemaphoreType.DMA,
            pltpu.SemaphoreType.DMA,
        ],
    )
    def k(xd_hbm, s_hbm, kj_hbm, ji_hbm, pm_hbm, bnd_hbm, agg_hbm,
          acc, kj_v, pm_v, li_v, ji_v, bnd_v, xrow_v, srow_v,
          sem1, sem2):
        c = lax.axis_index("c")
        s = lax.axis_index("s")
        pltpu.sync_copy(bnd_hbm, bnd_v)
        iota16 = lax.iota(_i32, 16)

        def _bnd(ix):
            q = pl.multiple_of(ix // 16 * 16, 16)
            b16 = bnd_v[pl.ds(q, 16)]
            return jnp.sum(jnp.where(iota16 == ix % 16, b16, 0))

        def chunk_body(kk, carry):
            ch = c + _NC * kk
            base_e = ch * _CHUNK_E
            rows_c = jnp.minimum(_CHUNK_E, _E - base_e)
            t0 = _bnd(ch)
            t1 = _bnd(ch + 1)
            _zero_vmem(xrow_v, _CT, _IE)
            z0 = pl.multiple_of(s * 640, 8)
            pltpu.sync_copy(xrow_v, acc.at[pl.ds(z0, _CT)])
            pltpu.sync_copy(xrow_v.at[pl.ds(0, 128)],
                            acc.at[pl.ds(pl.multiple_of(z0 + _CT, 8), 128)])

            @pl.when(s == 0)
            def _():
                pltpu.sync_copy(xrow_v.at[pl.ds(0, 16)],
                                acc.at[pl.ds(_CHUNK_E, 16)])

            plsc.subcore_barrier()
            span = t1 - t0
            st = t0 + span * s // _NS
            en = t0 + span * (s + 1) // _NS
            sal = (st // 16) * 16
            nsub = (en - sal + _CT - 1) // _CT

            def sub_body(m, carry2):
                t = pl.multiple_of(sal + m * _CT, 8)
                pltpu.sync_copy(kj_hbm.at[pl.ds(t, _CT)], kj_v)
                pltpu.sync_copy(pm_hbm.at[pl.ds(t, _CT)], pm_v)
                pltpu.sync_copy(ji_hbm.at[pl.ds(t, _CT)], ji_v)
                gx = pltpu.async_copy(xd_hbm.at[kj_v], xrow_v, sem1)
                gs = pltpu.async_copy(s_hbm.at[pm_v], srow_v, sem2)
                gx.wait()
                gs.wait()

                def liq(q, carry3):
                    pos = t + q * 16 + iota16
                    ji16 = ji_v[pl.ds(q * 16, 16)]
                    ok = (pos >= st) & (pos < en)
                    li_v[pl.ds(q * 16, 16)] = jnp.where(
                        ok, ji16 - base_e, _CHUNK_E)
                    return carry3

                lax.fori_loop(0, _CT // 16, liq, None)

                def mulq(r, carry3):
                    for rr in range(4):
                        for cc in range(_IE // 16):
                            sl = pl.ds(cc * 16, 16)
                            row = r * 4 + rr
                            xrow_v[row, sl] = xrow_v[row, sl] * srow_v[row, sl]
                    return carry3

                lax.fori_loop(0, _CT // 4, mulq, None)
                pltpu.sync_copy(xrow_v, acc.at[li_v], add=True)
                return carry2

            lax.fori_loop(0, nsub, sub_body, None)
            plsc.subcore_barrier()
            rpt = rows_c // _NS

            def wr(q, carry2):
                off = pl.multiple_of(s * rpt + q * 16, 8)
                pltpu.sync_copy(acc.at[pl.ds(off, 16)],
                                agg_hbm.at[pl.ds(pl.multiple_of(
                                    base_e + off, 8), 16)])
                return carry2

            lax.fori_loop(0, rpt // 16, wr, None)
            plsc.subcore_barrier()
            return carry

        lax.fori_loop(0, _NCHUNK // _NC, chunk_body, None)

    return k(xd, s_tab, skj_p, sji_p, perm_p, bounds)


def kernel(z, rbf, sbf, i, j, idx_kj, idx_ji, batch, params):
    z = z.astype(_i32)
    i = i.astype(_i32)
    j = j.astype(_i32)
    idx_kj = idx_kj.astype(_i32)
    idx_ji = idx_ji.astype(_i32)
    batch = batch.astype(_i32)

    perm = jnp.argsort(idx_ji).astype(_i32)
    sji = jnp.take(idx_ji, perm)
    skj = jnp.take(idx_kj, perm)
    chunk_edges = jnp.arange(_NCHUNK + 1, dtype=_i32) * _CHUNK_E
    bounds = jnp.searchsorted(sji, chunk_edges, side='left').astype(_i32)
    bounds = jnp.concatenate(
        [bounds, jnp.zeros((32 - _NCHUNK - 1,), _i32)])
    padi = jnp.zeros((_CT,), _i32)
    sji_p = jnp.concatenate([sji, jnp.full((_CT,), _E, _i32)])
    skj_p = jnp.concatenate([skj, padi])
    perm_p = jnp.concatenate([perm, padi])

    z2d = z.reshape(_N, 1)
    batch2d = batch.reshape(_N, 1)

    P = params
    emb = jnp.zeros((128, _HC), _f32).at[:95].set(P['emb_table'])
    I = P['init']
    W0 = I['lin_rbf_0']['w']
    b0 = I['lin_rbf_0']['b'].reshape(1, _HC)
    Wfull = I['lin']['w']
    Wa, Wb, Wc = Wfull[:_HC], Wfull[_HC:2 * _HC], Wfull[2 * _HC:]
    bc = I['lin']['b'].reshape(1, _HC)
    W1 = I['lin_rbf_1']['w']

    x = _k_emb(z2d, emb)
    xi, xj = _sc_gather_xij(x, i, j)
    e1, e2 = _k_init(xi, xj, rbf, W0, b0, Wa, Wb, Wc, bc, W1)

    def update_v(V, e2v):
        vparts = _sc_segsum_v(e2v, i)
        return _k_out(
            vparts[:_N], vparts[_N:], batch2d,
            V['lin_up']['w'], V['lin_up']['b'].reshape(1, _OE),
            V['lins'][0]['w'], V['lins'][0]['b'].reshape(1, _OE),
            V['lins'][1]['w'], V['lins'][1]['b'].reshape(1, _OE),
            V['lins'][2]['w'], V['lins'][2]['b'].reshape(1, _OE),
            V['lin']['w'])

    u = update_v(P['update_v'][0], e2)

    for li in range(_NL):
        L = P['update_e'][li]
        Wr = jnp.dot(L['lin_rbf1']['w'], L['lin_rbf2']['w'])
        Ws = jnp.dot(L['lin_sbf1']['w'], L['lin_sbf2']['w'])
        xji, xd = _k_pre(
            e1, rbf,
            L['lin_ji']['w'], L['lin_ji']['b'].reshape(1, _HC),
            L['lin_kj']['w'], L['lin_kj']['b'].reshape(1, _HC),
            Wr, L['lin_down']['w'])
        s_tab = _k_s(sbf, Ws)
        agg = _sc_triplet(xd, s_tab, skj_p, sji_p, perm_p, bounds)
        e1, e2 = _k_post(
            agg, xji, e1, rbf,
            L['lin_up']['w'],
            L['before'][0]['lin1']['w'], L['before'][0]['lin1']['b'].reshape(1, _HC),
            L['before'][0]['lin2']['w'], L['before'][0]['lin2']['b'].reshape(1, _HC),
            L['lin']['w'], L['lin']['b'].reshape(1, _HC),
            L['after'][0]['lin1']['w'], L['after'][0]['lin1']['b'].reshape(1, _HC),
            L['after'][0]['lin2']['w'], L['after'][0]['lin2']['b'].reshape(1, _HC),
            L['after'][1]['lin1']['w'], L['after'][1]['lin1']['b'].reshape(1, _HC),
            L['after'][1]['lin2']['w'], L['after'][1]['lin2']['b'].reshape(1, _HC),
            L['lin_rbf']['w'])
        u = u + update_v(P['update_v'][li + 1], e2)

    return u


# speedup vs baseline: 6.2199x; 1.0154x over previous
import functools

import jax
import jax.numpy as jnp
from jax import lax
from jax.experimental import pallas as pl
from jax.experimental.pallas import tpu as pltpu
from jax.experimental.pallas import tpu_sc as plsc

_N = 10000
_E = 160000
_T = 640000
_NG = 64
_HC = 128
_IE = 64
_NR = 6
_SBF = 42
_OE = 256
_NL = 4

_NC = 2
_NS = 16

_BLK_E = 640
_BLK_T = 1024
_BLK_N = 400
_CHUNK_E = 10240
_NCHUNK = 16
_CT = 256
_CG = 200

_f32 = jnp.float32
_i32 = jnp.int32


def _swish(x):
    return x * (1.0 / (1.0 + jnp.exp(-x)))


def _full_spec(a):
    nd = a.ndim
    return pl.BlockSpec(a.shape, lambda b, _nd=nd: (0,) * _nd)


def _row_spec(blk, d):
    return pl.BlockSpec((blk, d), lambda b: (b, 0))


def _edense(body, grid, data_specs, weights, out_shapes, out_specs, *data):
    in_specs = list(data_specs) + [_full_spec(w) for w in weights]
    return pl.pallas_call(
        body,
        grid=grid,
        in_specs=in_specs,
        out_specs=out_specs,
        out_shape=out_shapes,
    )(*data, *weights)


def _k_emb(z2d, table):
    def body(z_ref, t_ref, o_ref):
        z = z_ref[...]
        oh = (z == lax.broadcasted_iota(_i32, (1, 128), 1)).astype(_f32)
        o_ref[...] = jnp.dot(oh, t_ref[...], preferred_element_type=_f32)

    return _edense(
        body, (_N // _BLK_N,), [_row_spec(_BLK_N, 1)], [table],
        jax.ShapeDtypeStruct((_N, _HC), _f32), _row_spec(_BLK_N, _HC), z2d)


def _k_init(xi, xj, rbf, W0, b0, Wa, Wb, Wc, bc, W1):
    def body(xi_ref, xj_ref, rbf_ref, W0r, b0r, War, Wbr, Wcr, bcr, W1r,
             e1_ref, e2_ref):
        rbf_v = rbf_ref[...]
        rbf0 = _swish(jnp.dot(rbf_v, W0r[...], preferred_element_type=_f32)
                      + b0r[...])
        e1 = _swish(jnp.dot(xi_ref[...], War[...], preferred_element_type=_f32)
                    + jnp.dot(xj_ref[...], Wbr[...], preferred_element_type=_f32)
                    + jnp.dot(rbf0, Wcr[...], preferred_element_type=_f32)
                    + bcr[...])
        e1_ref[...] = e1
        e2_ref[...] = jnp.dot(rbf_v, W1r[...], preferred_element_type=_f32) * e1

    return _edense(
        body, (_E // _BLK_E,),
        [_row_spec(_BLK_E, _HC), _row_spec(_BLK_E, _HC), _row_spec(_BLK_E, _NR)],
        [W0, b0, Wa, Wb, Wc, bc, W1],
        [jax.ShapeDtypeStruct((_E, _HC), _f32)] * 2,
        [_row_spec(_BLK_E, _HC)] * 2,
        xi, xj, rbf)


def _k_pre(e1, rbf, Wji, bji, Wkj, bkj, Wr, Wd):
    def body(x_ref, rbf_ref, Wjir, bjir, Wkjr, bkjr, Wrr, Wdr,
             xji_ref, xd_ref):
        x1 = x_ref[...]
        xji_ref[...] = _swish(
            jnp.dot(x1, Wjir[...], preferred_element_type=_f32) + bjir[...])
        xkj = _swish(
            jnp.dot(x1, Wkjr[...], preferred_element_type=_f32) + bkjr[...])
        r = jnp.dot(rbf_ref[...], Wrr[...], preferred_element_type=_f32)
        xd_ref[...] = _swish(
            jnp.dot(xkj * r, Wdr[...], preferred_element_type=_f32))

    return _edense(
        body, (_E // _BLK_E,),
        [_row_spec(_BLK_E, _HC), _row_spec(_BLK_E, _NR)],
        [Wji, bji, Wkj, bkj, Wr, Wd],
        [jax.ShapeDtypeStruct((_E, _HC), _f32),
         jax.ShapeDtypeStruct((_E, _IE), _f32)],
        [_row_spec(_BLK_E, _HC), _row_spec(_BLK_E, _IE)],
        e1, rbf)


def _k_s(sbf, Ws):
    def body(s_ref, Wr, o_ref):
        o_ref[...] = jnp.dot(s_ref[...], Wr[...], preferred_element_type=_f32)

    return _edense(
        body, (_T // _BLK_T,), [_row_spec(_BLK_T, _SBF)], [Ws],
        jax.ShapeDtypeStruct((_T, _IE), _f32), _row_spec(_BLK_T, _IE), sbf)


def _k_post(agg, xji, x1, rbf, Wup, Wb1, bb1, Wb2, bb2, Wl, bl,
            Wa11, ba11, Wa12, ba12, Wa21, ba21, Wa22, ba22, Wrbf):
    def body(agg_ref, xji_ref, x1_ref, rbf_ref, Wupr, Wb1r, bb1r, Wb2r, bb2r,
             Wlr, blr, Wa11r, ba11r, Wa12r, ba12r, Wa21r, ba21r, Wa22r, ba22r,
             Wrbfr, e1_ref, e2_ref):
        def linr(v, w, b):
            return jnp.dot(v, w[...], preferred_element_type=_f32) + b[...]

        xup = _swish(jnp.dot(agg_ref[...], Wupr[...],
                             preferred_element_type=_f32))
        e1 = xji_ref[...] + xup
        e1 = e1 + _swish(linr(_swish(linr(e1, Wb1r, bb1r)), Wb2r, bb2r))
        e1 = _swish(linr(e1, Wlr, blr)) + x1_ref[...]
        e1 = e1 + _swish(linr(_swish(linr(e1, Wa11r, ba11r)), Wa12r, ba12r))
        e1 = e1 + _swish(linr(_swish(linr(e1, Wa21r, ba21r)), Wa22r, ba22r))
        e1_ref[...] = e1
        e2_ref[...] = jnp.dot(rbf_ref[...], Wrbfr[...],
                              preferred_element_type=_f32) * e1

    return _edense(
        body, (_E // _BLK_E,),
        [_row_spec(_BLK_E, _IE), _row_spec(_BLK_E, _HC),
         _row_spec(_BLK_E, _HC), _row_spec(_BLK_E, _NR)],
        [Wup, Wb1, bb1, Wb2, bb2, Wl, bl, Wa11, ba11, Wa12, ba12,
         Wa21, ba21, Wa22, ba22, Wrbf],
        [jax.ShapeDtypeStruct((_E, _HC), _f32)] * 2,
        [_row_spec(_BLK_E, _HC)] * 2,
        agg, xji, x1, rbf)


def _k_out(va, vb, batch2d, Wup, bup, W1, b1, W2, b2, W3, b3, Wf):
    def body(va_ref, vb_ref, bt_ref, Wupr, bupr, W1r, b1r, W2r, b2r, W3r, b3r,
             Wfr, u_ref):
        v = va_ref[...] + vb_ref[...]
        v = jnp.dot(v, Wupr[...], preferred_element_type=_f32) + bupr[...]
        v = _swish(jnp.dot(v, W1r[...], preferred_element_type=_f32) + b1r[...])
        v = _swish(jnp.dot(v, W2r[...], preferred_element_type=_f32) + b2r[...])
        v = _swish(jnp.dot(v, W3r[...], preferred_element_type=_f32) + b3r[...])
        vo = jnp.dot(v, Wfr[...], preferred_element_type=_f32)
        oh = (bt_ref[...] == lax.broadcasted_iota(_i32, (1, _NG), 1)
              ).astype(_f32)
        u = lax.dot_general(oh, vo, (((0,), (0,)), ((), ())),
                            preferred_element_type=_f32)

        @pl.when(pl.program_id(0) == 0)
        def _():
            u_ref[...] = jnp.zeros_like(u_ref)

        u_ref[...] += u

    return _edense(
        body, (_N // _BLK_N,),
        [_row_spec(_BLK_N, _HC), _row_spec(_BLK_N, _HC), _row_spec(_BLK_N, 1)],
        [Wup, bup, W1, b1, W2, b2, W3, b3, Wf],
        jax.ShapeDtypeStruct((_NG, 1), _f32),
        pl.BlockSpec((_NG, 1), lambda b: (0, 0)),
        va, vb, batch2d)


def _sc_mesh():
    return plsc.VectorSubcoreMesh(core_axis_name="c", subcore_axis_name="s",
                                  num_cores=_NC, num_subcores=_NS)


def _zero_vmem(ref, rows, cols):
    z16 = jnp.zeros((16,), _f32)

    def body(r, carry):
        for cc in range(cols // 16):
            ref[r, pl.ds(cc * 16, 16)] = z16
        return carry

    lax.fori_loop(0, rows, body, None)


def _sc_gather_xij(x, iarr, jarr):
    per_tec = _E // (_NC * _NS)

    @functools.partial(
        pl.kernel,
        out_type=[jax.ShapeDtypeStruct((_E, _HC), _f32)] * 2,
        mesh=_sc_mesh(),
        compiler_params=pltpu.CompilerParams(needs_layout_passes=False, use_tc_tiling_on_sc=False),
        scratch_types=[
            pltpu.MemorySpace.VMEM((_CG,), _i32),
            pltpu.MemorySpace.VMEM((_CG, _HC), _f32),
            pltpu.SemaphoreType.DMA,
        ],
    )
    def k(x_hbm, i_hbm, j_hbm, xi_hbm, xj_hbm, idx_v, row_v, sem):
        c = lax.axis_index("c")
        s = lax.axis_index("s")
        base = (c * _NS + s) * per_tec

        def body(kk, carry):
            t = pl.multiple_of(base + kk * _CG, 8)
            pltpu.sync_copy(i_hbm.at[pl.ds(t, _CG)], idx_v)
            pltpu.async_copy(x_hbm.at[idx_v], row_v, sem).wait()
            pltpu.sync_copy(row_v, xi_hbm.at[pl.ds(t, _CG)])
            pltpu.sync_copy(j_hbm.at[pl.ds(t, _CG)], idx_v)
            pltpu.async_copy(x_hbm.at[idx_v], row_v, sem).wait()
            pltpu.sync_copy(row_v, xj_hbm.at[pl.ds(t, _CG)])
            return carry

        lax.fori_loop(0, per_tec // _CG, body, None)

    return k(x, iarr, jarr)


def _sc_segsum_v(e2, iarr):
    per_tec = _E // (_NC * _NS)
    zrows = 80

    @functools.partial(
        pl.kernel,
        out_type=jax.ShapeDtypeStruct((_NC * _N, _HC), _f32),
        mesh=_sc_mesh(),
        compiler_params=pltpu.CompilerParams(needs_layout_passes=False, use_tc_tiling_on_sc=False),
        scratch_types=[
            pltpu.MemorySpace.VMEM_SHARED((_N, _HC), _f32),
            pltpu.MemorySpace.VMEM((_CG,), _i32),
            pltpu.MemorySpace.VMEM((_CG, _HC), _f32),
            pltpu.MemorySpace.VMEM((zrows, _HC), _f32),
        ],
    )
    def k(e2_hbm, i_hbm, out_hbm, acc, idx_v, row_v, zbuf):
        c = lax.axis_index("c")
        s = lax.axis_index("s")
        _zero_vmem(zbuf, zrows, _HC)
        nz = jnp.where(s == 15, 400 // zrows, 640 // zrows)

        def zb(q, carry):
            off = pl.multiple_of(s * 640 + q * zrows, 8)
            pltpu.sync_copy(zbuf, acc.at[pl.ds(off, zrows)])
            return carry

        lax.fori_loop(0, nz, zb, None)
        plsc.subcore_barrier()

        base = c * (_E // _NC) + s * per_tec

        def body(kk, carry):
            t = pl.multiple_of(base + kk * _CG, 8)
            pltpu.sync_copy(i_hbm.at[pl.ds(t, _CG)], idx_v)
            pltpu.sync_copy(e2_hbm.at[pl.ds(t, _CG)], row_v)
            pltpu.sync_copy(row_v, acc.at[idx_v], add=True)
            return carry

        lax.fori_loop(0, per_tec // _CG, body, None)
        plsc.subcore_barrier()

        def wr(q, carry):
            off = pl.multiple_of(s * 640 + q * zrows, 8)
            pltpu.sync_copy(acc.at[pl.ds(off, zrows)],
                            out_hbm.at[pl.ds(pl.multiple_of(
                                c * _N + off, 8), zrows)])
            return carry

        lax.fori_loop(0, nz, wr, None)

    return k(e2, iarr)


def _sc_triplet(xd, s_tab, skj_p, sji_p, perm_p, bounds):
    @functools.partial(
        pl.kernel,
        out_type=jax.ShapeDtypeStruct((_E, _IE), _f32),
        mesh=_sc_mesh(),
        compiler_params=pltpu.CompilerParams(needs_layout_passes=False, use_tc_tiling_on_sc=False),
        scratch_types=[
            pltpu.MemorySpace.VMEM_SHARED((_CHUNK_E + 16, _IE), _f32),
            [pltpu.MemorySpace.VMEM((_CT,), _i32)] * 2,
            [pltpu.MemorySpace.VMEM((_CT,), _i32)] * 2,
            [pltpu.MemorySpace.VMEM((_CT,), _i32)] * 2,
            pltpu.MemorySpace.VMEM((_CT,), _i32),
            pltpu.MemorySpace.VMEM((32,), _i32),
            [pltpu.MemorySpace.VMEM((_CT, _IE), _f32)] * 2,
            [pltpu.MemorySpace.VMEM((_CT, _IE), _f32)] * 2,
            [pltpu.SemaphoreType.DMA] * 2,
            [pltpu.SemaphoreType.DMA] * 2,
        ],
    )
    def k(xd_hbm, s_hbm, kj_hbm, ji_hbm, pm_hbm, bnd_hbm, agg_hbm,
          acc, kj_v, pm_v, ji_v, li_v, bnd_v, xrow_v, srow_v,
          semx, sems):
        c = lax.axis_index("c")
        s = lax.axis_index("s")
        pltpu.sync_copy(bnd_hbm, bnd_v)
        iota16 = lax.iota(_i32, 16)

        def _bnd(ix):
            q = pl.multiple_of(ix // 16 * 16, 16)
            b16 = bnd_v[pl.ds(q, 16)]
            return jnp.sum(jnp.where(iota16 == ix % 16, b16, 0))

        def chunk_body(kk, carry):
            ch = c + _NC * kk
            base_e = ch * _CHUNK_E
            rows_c = jnp.minimum(_CHUNK_E, _E - base_e)
            t0 = _bnd(ch)
            t1 = _bnd(ch + 1)
            _zero_vmem(xrow_v[0], _CT, _IE)
            z0 = pl.multiple_of(s * 640, 8)
            pltpu.sync_copy(xrow_v[0], acc.at[pl.ds(z0, _CT)])
            pltpu.sync_copy(xrow_v[0], acc.at[pl.ds(
                pl.multiple_of(z0 + _CT, 8), _CT)])
            pltpu.sync_copy(xrow_v[0].at[pl.ds(0, 128)],
                            acc.at[pl.ds(pl.multiple_of(z0 + 2 * _CT, 8), 128)])

            @pl.when(s == 0)
            def _():
                pltpu.sync_copy(xrow_v[0].at[pl.ds(0, 16)],
                                acc.at[pl.ds(_CHUNK_E, 16)])

            plsc.subcore_barrier()
            span = t1 - t0
            st = t0 + span * s // _NS
            en = t0 + span * (s + 1) // _NS
            sal = (st // 16) * 16
            nsub = (en - sal + _CT - 1) // _CT

            def fire(m, b):
                t = pl.multiple_of(sal + m * _CT, 8)
                pltpu.sync_copy(kj_hbm.at[pl.ds(t, _CT)], kj_v[b])
                pltpu.sync_copy(pm_hbm.at[pl.ds(t, _CT)], pm_v[b])
                pltpu.sync_copy(ji_hbm.at[pl.ds(t, _CT)], ji_v[b])
                pltpu.async_copy(xd_hbm.at[kj_v[b]], xrow_v[b], semx[b])
                pltpu.async_copy(s_hbm.at[pm_v[b]], srow_v[b], sems[b])

            @pl.when(nsub > 0)
            def _():
                fire(0, 0)

            def consume(m, b):
                t = pl.multiple_of(sal + m * _CT, 8)

                @pl.when(m + 1 < nsub)
                def _():
                    fire(m + 1, 1 - b)

                pltpu.make_async_copy(
                    xd_hbm.at[kj_v[b]], xrow_v[b], semx[b]).wait()
                pltpu.make_async_copy(
                    s_hbm.at[pm_v[b]], srow_v[b], sems[b]).wait()

                def liq(q, carry3):
                    pos = t + q * 16 + iota16
                    ji16 = ji_v[b][pl.ds(q * 16, 16)]
                    ok = (pos >= st) & (pos < en)
                    li_v[pl.ds(q * 16, 16)] = jnp.where(
                        ok, ji16 - base_e, _CHUNK_E)
                    return carry3

                lax.fori_loop(0, _CT // 16, liq, None)

                def mulq(r, carry3):
                    for rr in range(4):
                        for cc in range(_IE // 16):
                            sl = pl.ds(cc * 16, 16)
                            row = r * 4 + rr
                            xrow_v[b][row, sl] = (xrow_v[b][row, sl]
                                                  * srow_v[b][row, sl])
                    return carry3

                lax.fori_loop(0, _CT // 4, mulq, None)
                pltpu.sync_copy(xrow_v[b], acc.at[li_v], add=True)

            def sub_body(m, carry2):
                @pl.when(m % 2 == 0)
                def _():
                    consume(m, 0)

                @pl.when(m % 2 == 1)
                def _():
                    consume(m, 1)

                return carry2

            lax.fori_loop(0, nsub, sub_body, None)
            plsc.subcore_barrier()
            rpt = rows_c // _NS

            def wr(q, carry2):
                off = pl.multiple_of(s * rpt + q * 16, 8)
                pltpu.sync_copy(acc.at[pl.ds(off, 16)],
                                agg_hbm.at[pl.ds(pl.multiple_of(
                                    base_e + off, 8), 16)])
                return carry2

            lax.fori_loop(0, rpt // 16, wr, None)
            plsc.subcore_barrier()
            return carry

        lax.fori_loop(0, _NCHUNK // _NC, chunk_body, None)

    return k(xd, s_tab, skj_p, sji_p, perm_p, bounds)


def kernel(z, rbf, sbf, i, j, idx_kj, idx_ji, batch, params):
    z = z.astype(_i32)
    i = i.astype(_i32)
    j = j.astype(_i32)
    idx_kj = idx_kj.astype(_i32)
    idx_ji = idx_ji.astype(_i32)
    batch = batch.astype(_i32)

    perm = jnp.argsort(idx_ji).astype(_i32)
    sji = jnp.take(idx_ji, perm)
    skj = jnp.take(idx_kj, perm)
    chunk_edges = jnp.arange(_NCHUNK + 1, dtype=_i32) * _CHUNK_E
    bounds = jnp.searchsorted(sji, chunk_edges, side='left').astype(_i32)
    bounds = jnp.concatenate(
        [bounds, jnp.zeros((32 - _NCHUNK - 1,), _i32)])
    padi = jnp.zeros((_CT,), _i32)
    sji_p = jnp.concatenate([sji, jnp.full((_CT,), _E, _i32)])
    skj_p = jnp.concatenate([skj, padi])
    perm_p = jnp.concatenate([perm, padi])

    z2d = z.reshape(_N, 1)
    batch2d = batch.reshape(_N, 1)

    P = params
    emb = jnp.zeros((128, _HC), _f32).at[:95].set(P['emb_table'])
    I = P['init']
    W0 = I['lin_rbf_0']['w']
    b0 = I['lin_rbf_0']['b'].reshape(1, _HC)
    Wfull = I['lin']['w']
    Wa, Wb, Wc = Wfull[:_HC], Wfull[_HC:2 * _HC], Wfull[2 * _HC:]
    bc = I['lin']['b'].reshape(1, _HC)
    W1 = I['lin_rbf_1']['w']

    x = _k_emb(z2d, emb)
    xi, xj = _sc_gather_xij(x, i, j)
    e1, e2 = _k_init(xi, xj, rbf, W0, b0, Wa, Wb, Wc, bc, W1)

    def update_v(V, e2v):
        vparts = _sc_segsum_v(e2v, i)
        return _k_out(
            vparts[:_N], vparts[_N:], batch2d,
            V['lin_up']['w'], V['lin_up']['b'].reshape(1, _OE),
            V['lins'][0]['w'], V['lins'][0]['b'].reshape(1, _OE),
            V['lins'][1]['w'], V['lins'][1]['b'].reshape(1, _OE),
            V['lins'][2]['w'], V['lins'][2]['b'].reshape(1, _OE),
            V['lin']['w'])

    u = update_v(P['update_v'][0], e2)

    for li in range(_NL):
        L = P['update_e'][li]
        Wr = jnp.dot(L['lin_rbf1']['w'], L['lin_rbf2']['w'])
        Ws = jnp.dot(L['lin_sbf1']['w'], L['lin_sbf2']['w'])
        xji, xd = _k_pre(
            e1, rbf,
            L['lin_ji']['w'], L['lin_ji']['b'].reshape(1, _HC),
            L['lin_kj']['w'], L['lin_kj']['b'].reshape(1, _HC),
            Wr, L['lin_down']['w'])
        s_tab = _k_s(sbf, Ws)
        agg = _sc_triplet(xd, s_tab, skj_p, sji_p, perm_p, bounds)
        e1, e2 = _k_post(
            agg, xji, e1, rbf,
            L['lin_up']['w'],
            L['before'][0]['lin1']['w'], L['before'][0]['lin1']['b'].reshape(1, _HC),
            L['before'][0]['lin2']['w'], L['before'][0]['lin2']['b'].reshape(1, _HC),
            L['lin']['w'], L['lin']['b'].reshape(1, _HC),
            L['after'][0]['lin1']['w'], L['after'][0]['lin1']['b'].reshape(1, _HC),
            L['after'][0]['lin2']['w'], L['after'][0]['lin2']['b'].reshape(1, _HC),
            L['after'][1]['lin1']['w'], L['after'][1]['lin1']['b'].reshape(1, _HC),
            L['after'][1]['lin2']['w'], L['after'][1]['lin2']['b'].reshape(1, _HC),
            L['lin_rbf']['w'])
        u = u + update_v(P['update_v'][li + 1], e2)

    return u


# speedup vs baseline: 6.3577x; 1.0221x over previous
import functools

import jax
import jax.numpy as jnp
from jax import lax
from jax.experimental import pallas as pl
from jax.experimental.pallas import tpu as pltpu
from jax.experimental.pallas import tpu_sc as plsc

_N = 10000
_E = 160000
_T = 640000
_NG = 64
_HC = 128
_IE = 64
_NR = 6
_SBF = 42
_OE = 256
_NL = 4

_NC = 2
_NS = 16

_BLK_E = 640
_BLK_T = 1024
_BLK_N = 400
_CHUNK_E = 10240
_NCHUNK = 16
_CT = 256
_CG = 200

_f32 = jnp.float32
_i32 = jnp.int32


def _swish(x):
    return x * (1.0 / (1.0 + jnp.exp(-x)))


def _full_spec(a):
    nd = a.ndim
    return pl.BlockSpec(a.shape, lambda b, _nd=nd: (0,) * _nd)


def _row_spec(blk, d):
    return pl.BlockSpec((blk, d), lambda b: (b, 0))


def _edense(body, grid, data_specs, weights, out_shapes, out_specs, *data):
    in_specs = list(data_specs) + [_full_spec(w) for w in weights]
    return pl.pallas_call(
        body,
        grid=grid,
        in_specs=in_specs,
        out_specs=out_specs,
        out_shape=out_shapes,
    )(*data, *weights)


def _k_emb(z2d, table):
    def body(z_ref, t_ref, o_ref):
        z = z_ref[...]
        oh = (z == lax.broadcasted_iota(_i32, (1, 128), 1)).astype(_f32)
        o_ref[...] = jnp.dot(oh, t_ref[...], preferred_element_type=_f32)

    return _edense(
        body, (_N // _BLK_N,), [_row_spec(_BLK_N, 1)], [table],
        jax.ShapeDtypeStruct((_N, _HC), _f32), _row_spec(_BLK_N, _HC), z2d)


def _k_init(xi, xj, rbf, W0, b0, Wa, Wb, Wc, bc, W1):
    def body(xi_ref, xj_ref, rbf_ref, W0r, b0r, War, Wbr, Wcr, bcr, W1r,
             e1_ref, e2_ref):
        rbf_v = rbf_ref[...]
        rbf0 = _swish(jnp.dot(rbf_v, W0r[...], preferred_element_type=_f32)
                      + b0r[...])
        e1 = _swish(jnp.dot(xi_ref[...], War[...], preferred_element_type=_f32)
                    + jnp.dot(xj_ref[...], Wbr[...], preferred_element_type=_f32)
                    + jnp.dot(rbf0, Wcr[...], preferred_element_type=_f32)
                    + bcr[...])
        e1_ref[...] = e1
        e2_ref[...] = jnp.dot(rbf_v, W1r[...], preferred_element_type=_f32) * e1

    return _edense(
        body, (_E // _BLK_E,),
        [_row_spec(_BLK_E, _HC), _row_spec(_BLK_E, _HC), _row_spec(_BLK_E, _NR)],
        [W0, b0, Wa, Wb, Wc, bc, W1],
        [jax.ShapeDtypeStruct((_E, _HC), _f32)] * 2,
        [_row_spec(_BLK_E, _HC)] * 2,
        xi, xj, rbf)


def _k_pre(e1, rbf, Wji, bji, Wkj, bkj, Wr, Wd):
    def body(x_ref, rbf_ref, Wjir, bjir, Wkjr, bkjr, Wrr, Wdr,
             xji_ref, xd_ref):
        x1 = x_ref[...]
        xji_ref[...] = _swish(
            jnp.dot(x1, Wjir[...], preferred_element_type=_f32) + bjir[...])
        xkj = _swish(
            jnp.dot(x1, Wkjr[...], preferred_element_type=_f32) + bkjr[...])
        r = jnp.dot(rbf_ref[...], Wrr[...], preferred_element_type=_f32)
        xd_ref[...] = _swish(
            jnp.dot(xkj * r, Wdr[...], preferred_element_type=_f32))

    return _edense(
        body, (_E // _BLK_E,),
        [_row_spec(_BLK_E, _HC), _row_spec(_BLK_E, _NR)],
        [Wji, bji, Wkj, bkj, Wr, Wd],
        [jax.ShapeDtypeStruct((_E, _HC), _f32),
         jax.ShapeDtypeStruct((_E, _IE), _f32)],
        [_row_spec(_BLK_E, _HC), _row_spec(_BLK_E, _IE)],
        e1, rbf)


def _k_s(sbf, Ws):
    def body(s_ref, Wr, o_ref):
        o_ref[...] = jnp.dot(s_ref[...], Wr[...], preferred_element_type=_f32)

    return _edense(
        body, (_T // _BLK_T,), [_row_spec(_BLK_T, _SBF)], [Ws],
        jax.ShapeDtypeStruct((_T, _IE), _f32), _row_spec(_BLK_T, _IE), sbf)


def _k_post(agg, xji, x1, rbf, Wup, Wb1, bb1, Wb2, bb2, Wl, bl,
            Wa11, ba11, Wa12, ba12, Wa21, ba21, Wa22, ba22, Wrbf):
    def body(agg_ref, xji_ref, x1_ref, rbf_ref, Wupr, Wb1r, bb1r, Wb2r, bb2r,
             Wlr, blr, Wa11r, ba11r, Wa12r, ba12r, Wa21r, ba21r, Wa22r, ba22r,
             Wrbfr, e1_ref, e2_ref):
        def linr(v, w, b):
            return jnp.dot(v, w[...], preferred_element_type=_f32) + b[...]

        xup = _swish(jnp.dot(agg_ref[...], Wupr[...],
                             preferred_element_type=_f32))
        e1 = xji_ref[...] + xup
        e1 = e1 + _swish(linr(_swish(linr(e1, Wb1r, bb1r)), Wb2r, bb2r))
        e1 = _swish(linr(e1, Wlr, blr)) + x1_ref[...]
        e1 = e1 + _swish(linr(_swish(linr(e1, Wa11r, ba11r)), Wa12r, ba12r))
        e1 = e1 + _swish(linr(_swish(linr(e1, Wa21r, ba21r)), Wa22r, ba22r))
        e1_ref[...] = e1
        e2_ref[...] = jnp.dot(rbf_ref[...], Wrbfr[...],
                              preferred_element_type=_f32) * e1

    return _edense(
        body, (_E // _BLK_E,),
        [_row_spec(_BLK_E, _IE), _row_spec(_BLK_E, _HC),
         _row_spec(_BLK_E, _HC), _row_spec(_BLK_E, _NR)],
        [Wup, Wb1, bb1, Wb2, bb2, Wl, bl, Wa11, ba11, Wa12, ba12,
         Wa21, ba21, Wa22, ba22, Wrbf],
        [jax.ShapeDtypeStruct((_E, _HC), _f32)] * 2,
        [_row_spec(_BLK_E, _HC)] * 2,
        agg, xji, x1, rbf)


def _k_out(va, vb, batch2d, Wup, bup, W1, b1, W2, b2, W3, b3, Wf):
    def body(va_ref, vb_ref, bt_ref, Wupr, bupr, W1r, b1r, W2r, b2r, W3r, b3r,
             Wfr, u_ref):
        v = va_ref[...] + vb_ref[...]
        v = jnp.dot(v, Wupr[...], preferred_element_type=_f32) + bupr[...]
        v = _swish(jnp.dot(v, W1r[...], preferred_element_type=_f32) + b1r[...])
        v = _swish(jnp.dot(v, W2r[...], preferred_element_type=_f32) + b2r[...])
        v = _swish(jnp.dot(v, W3r[...], preferred_element_type=_f32) + b3r[...])
        vo = jnp.dot(v, Wfr[...], preferred_element_type=_f32)
        oh = (bt_ref[...] == lax.broadcasted_iota(_i32, (1, _NG), 1)
              ).astype(_f32)
        u = lax.dot_general(oh, vo, (((0,), (0,)), ((), ())),
                            preferred_element_type=_f32)

        @pl.when(pl.program_id(0) == 0)
        def _():
            u_ref[...] = jnp.zeros_like(u_ref)

        u_ref[...] += u

    return _edense(
        body, (_N // _BLK_N,),
        [_row_spec(_BLK_N, _HC), _row_spec(_BLK_N, _HC), _row_spec(_BLK_N, 1)],
        [Wup, bup, W1, b1, W2, b2, W3, b3, Wf],
        jax.ShapeDtypeStruct((_NG, 1), _f32),
        pl.BlockSpec((_NG, 1), lambda b: (0, 0)),
        va, vb, batch2d)


def _sc_mesh():
    return plsc.VectorSubcoreMesh(core_axis_name="c", subcore_axis_name="s",
                                  num_cores=_NC, num_subcores=_NS)


def _zero_vmem(ref, rows, cols):
    z16 = jnp.zeros((16,), _f32)

    def body(r, carry):
        for cc in range(cols // 16):
            ref[r, pl.ds(cc * 16, 16)] = z16
        return carry

    lax.fori_loop(0, rows, body, None)


def _sc_gather_xij(x, iarr, jarr):
    per_tec = _E // (_NC * _NS)

    @functools.partial(
        pl.kernel,
        out_type=[jax.ShapeDtypeStruct((_E, _HC), _f32)] * 2,
        mesh=_sc_mesh(),
        compiler_params=pltpu.CompilerParams(needs_layout_passes=False, use_tc_tiling_on_sc=False),
        scratch_types=[
            pltpu.MemorySpace.VMEM((_CG,), _i32),
            pltpu.MemorySpace.VMEM((_CG, _HC), _f32),
            pltpu.SemaphoreType.DMA,
        ],
    )
    def k(x_hbm, i_hbm, j_hbm, xi_hbm, xj_hbm, idx_v, row_v, sem):
        c = lax.axis_index("c")
        s = lax.axis_index("s")
        base = (c * _NS + s) * per_tec

        def body(kk, carry):
            t = pl.multiple_of(base + kk * _CG, 8)
            pltpu.sync_copy(i_hbm.at[pl.ds(t, _CG)], idx_v)
            pltpu.async_copy(x_hbm.at[idx_v], row_v, sem).wait()
            pltpu.sync_copy(row_v, xi_hbm.at[pl.ds(t, _CG)])
            pltpu.sync_copy(j_hbm.at[pl.ds(t, _CG)], idx_v)
            pltpu.async_copy(x_hbm.at[idx_v], row_v, sem).wait()
            pltpu.sync_copy(row_v, xj_hbm.at[pl.ds(t, _CG)])
            return carry

        lax.fori_loop(0, per_tec // _CG, body, None)

    return k(x, iarr, jarr)


def _sc_segsum_v(e2, iarr):
    per_tec = _E // (_NC * _NS)
    zrows = 80

    @functools.partial(
        pl.kernel,
        out_type=jax.ShapeDtypeStruct((_NC * _N, _HC), _f32),
        mesh=_sc_mesh(),
        compiler_params=pltpu.CompilerParams(needs_layout_passes=False, use_tc_tiling_on_sc=False),
        scratch_types=[
            pltpu.MemorySpace.VMEM_SHARED((_N, _HC), _f32),
            pltpu.MemorySpace.VMEM((_CG,), _i32),
            pltpu.MemorySpace.VMEM((_CG, _HC), _f32),
            pltpu.MemorySpace.VMEM((zrows, _HC), _f32),
        ],
    )
    def k(e2_hbm, i_hbm, out_hbm, acc, idx_v, row_v, zbuf):
        c = lax.axis_index("c")
        s = lax.axis_index("s")
        _zero_vmem(zbuf, zrows, _HC)
        nz = jnp.where(s == 15, 400 // zrows, 640 // zrows)

        def zb(q, carry):
            off = pl.multiple_of(s * 640 + q * zrows, 8)
            pltpu.sync_copy(zbuf, acc.at[pl.ds(off, zrows)])
            return carry

        lax.fori_loop(0, nz, zb, None)
        plsc.subcore_barrier()

        base = c * (_E // _NC) + s * per_tec

        def body(kk, carry):
            t = pl.multiple_of(base + kk * _CG, 8)
            pltpu.sync_copy(i_hbm.at[pl.ds(t, _CG)], idx_v)
            pltpu.sync_copy(e2_hbm.at[pl.ds(t, _CG)], row_v)
            pltpu.sync_copy(row_v, acc.at[idx_v], add=True)
            return carry

        lax.fori_loop(0, per_tec // _CG, body, None)
        plsc.subcore_barrier()

        @pl.when(s != 15)
        def _():
            off = pl.multiple_of(s * 640, 8)
            pltpu.sync_copy(acc.at[pl.ds(off, 640)],
                            out_hbm.at[pl.ds(pl.multiple_of(
                                c * _N + off, 8), 640)])

        @pl.when(s == 15)
        def _():
            off = pl.multiple_of(15 * 640, 8)
            pltpu.sync_copy(acc.at[pl.ds(off, 400)],
                            out_hbm.at[pl.ds(pl.multiple_of(
                                c * _N + off, 8), 400)])

    return k(e2, iarr)


def _sc_triplet(xd, s_tab, skj_p, sji_p, perm_p, bounds):
    @functools.partial(
        pl.kernel,
        out_type=jax.ShapeDtypeStruct((_E, _IE), _f32),
        mesh=_sc_mesh(),
        compiler_params=pltpu.CompilerParams(needs_layout_passes=False, use_tc_tiling_on_sc=False),
        scratch_types=[
            pltpu.MemorySpace.VMEM_SHARED((_CHUNK_E + 16, _IE), _f32),
            [pltpu.MemorySpace.VMEM((_CT,), _i32)] * 2,
            [pltpu.MemorySpace.VMEM((_CT,), _i32)] * 2,
            [pltpu.MemorySpace.VMEM((_CT,), _i32)] * 2,
            pltpu.MemorySpace.VMEM((_CT,), _i32),
            pltpu.MemorySpace.VMEM((32,), _i32),
            [pltpu.MemorySpace.VMEM((_CT, _IE), _f32)] * 2,
            [pltpu.MemorySpace.VMEM((_CT, _IE), _f32)] * 2,
            [pltpu.SemaphoreType.DMA] * 2,
            [pltpu.SemaphoreType.DMA] * 2,
        ],
    )
    def k(xd_hbm, s_hbm, kj_hbm, ji_hbm, pm_hbm, bnd_hbm, agg_hbm,
          acc, kj_v, pm_v, ji_v, li_v, bnd_v, xrow_v, srow_v,
          semx, sems):
        c = lax.axis_index("c")
        s = lax.axis_index("s")
        pltpu.sync_copy(bnd_hbm, bnd_v)
        iota16 = lax.iota(_i32, 16)

        def _bnd(ix):
            q = pl.multiple_of(ix // 16 * 16, 16)
            b16 = bnd_v[pl.ds(q, 16)]
            return jnp.sum(jnp.where(iota16 == ix % 16, b16, 0))

        def chunk_body(kk, carry):
            ch = c + _NC * kk
            base_e = ch * _CHUNK_E
            rows_c = jnp.minimum(_CHUNK_E, _E - base_e)
            t0 = _bnd(ch)
            t1 = _bnd(ch + 1)
            _zero_vmem(xrow_v[0], _CT, _IE)
            z0 = pl.multiple_of(s * 640, 8)
            pltpu.sync_copy(xrow_v[0], acc.at[pl.ds(z0, _CT)])
            pltpu.sync_copy(xrow_v[0], acc.at[pl.ds(
                pl.multiple_of(z0 + _CT, 8), _CT)])
            pltpu.sync_copy(xrow_v[0].at[pl.ds(0, 128)],
                            acc.at[pl.ds(pl.multiple_of(z0 + 2 * _CT, 8), 128)])

            @pl.when(s == 0)
            def _():
                pltpu.sync_copy(xrow_v[0].at[pl.ds(0, 16)],
                                acc.at[pl.ds(_CHUNK_E, 16)])

            plsc.subcore_barrier()
            span = t1 - t0
            st = t0 + span * s // _NS
            en = t0 + span * (s + 1) // _NS
            sal = (st // 16) * 16
            nsub = (en - sal + _CT - 1) // _CT

            def fire(m, b):
                t = pl.multiple_of(sal + m * _CT, 8)
                pltpu.sync_copy(kj_hbm.at[pl.ds(t, _CT)], kj_v[b])
                pltpu.sync_copy(pm_hbm.at[pl.ds(t, _CT)], pm_v[b])
                pltpu.sync_copy(ji_hbm.at[pl.ds(t, _CT)], ji_v[b])
                pltpu.async_copy(xd_hbm.at[kj_v[b]], xrow_v[b], semx[b])
                pltpu.async_copy(s_hbm.at[pm_v[b]], srow_v[b], sems[b])

            @pl.when(nsub > 0)
            def _():
                fire(0, 0)

            def consume(m, b):
                t = pl.multiple_of(sal + m * _CT, 8)

                @pl.when(m + 1 < nsub)
                def _():
                    fire(m + 1, 1 - b)

                pltpu.make_async_copy(
                    xd_hbm.at[kj_v[b]], xrow_v[b], semx[b]).wait()
                pltpu.make_async_copy(
                    s_hbm.at[pm_v[b]], srow_v[b], sems[b]).wait()

                def liq(q, carry3):
                    pos = t + q * 16 + iota16
                    ji16 = ji_v[b][pl.ds(q * 16, 16)]
                    ok = (pos >= st) & (pos < en)
                    li_v[pl.ds(q * 16, 16)] = jnp.where(
                        ok, ji16 - base_e, _CHUNK_E)
                    return carry3

                lax.fori_loop(0, _CT // 16, liq, None)

                def mulq(r, carry3):
                    for rr in range(4):
                        for cc in range(_IE // 16):
                            sl = pl.ds(cc * 16, 16)
                            row = r * 4 + rr
                            xrow_v[b][row, sl] = (xrow_v[b][row, sl]
                                                  * srow_v[b][row, sl])
                    return carry3

                lax.fori_loop(0, _CT // 4, mulq, None)
                pltpu.sync_copy(xrow_v[b], acc.at[li_v], add=True)

            def sub_body(m, carry2):
                @pl.when(m % 2 == 0)
                def _():
                    consume(m, 0)

                @pl.when(m % 2 == 1)
                def _():
                    consume(m, 1)

                return carry2

            lax.fori_loop(0, nsub, sub_body, None)
            plsc.subcore_barrier()
            full = _CHUNK_E // _NS
            tail = (_E - (_NCHUNK - 1) * _CHUNK_E) // _NS

            @pl.when(rows_c == _CHUNK_E)
            def _():
                off = pl.multiple_of(s * full, 8)
                pltpu.sync_copy(acc.at[pl.ds(off, full)],
                                agg_hbm.at[pl.ds(pl.multiple_of(
                                    base_e + off, 8), full)])

            @pl.when(rows_c != _CHUNK_E)
            def _():
                off = pl.multiple_of(s * tail, 8)
                pltpu.sync_copy(acc.at[pl.ds(off, tail)],
                                agg_hbm.at[pl.ds(pl.multiple_of(
                                    base_e + off, 8), tail)])

            plsc.subcore_barrier()
            return carry

        lax.fori_loop(0, _NCHUNK // _NC, chunk_body, None)

    return k(xd, s_tab, skj_p, sji_p, perm_p, bounds)


def kernel(z, rbf, sbf, i, j, idx_kj, idx_ji, batch, params):
    z = z.astype(_i32)
    i = i.astype(_i32)
    j = j.astype(_i32)
    idx_kj = idx_kj.astype(_i32)
    idx_ji = idx_ji.astype(_i32)
    batch = batch.astype(_i32)

    perm = jnp.argsort(idx_ji).astype(_i32)
    sji = jnp.take(idx_ji, perm)
    skj = jnp.take(idx_kj, perm)
    chunk_edges = jnp.arange(_NCHUNK + 1, dtype=_i32) * _CHUNK_E
    bounds = jnp.searchsorted(sji, chunk_edges, side='left').astype(_i32)
    bounds = jnp.concatenate(
        [bounds, jnp.zeros((32 - _NCHUNK - 1,), _i32)])
    padi = jnp.zeros((_CT,), _i32)
    sji_p = jnp.concatenate([sji, jnp.full((_CT,), _E, _i32)])
    skj_p = jnp.concatenate([skj, padi])
    perm_p = jnp.concatenate([perm, padi])

    z2d = z.reshape(_N, 1)
    batch2d = batch.reshape(_N, 1)

    P = params
    emb = jnp.zeros((128, _HC), _f32).at[:95].set(P['emb_table'])
    I = P['init']
    W0 = I['lin_rbf_0']['w']
    b0 = I['lin_rbf_0']['b'].reshape(1, _HC)
    Wfull = I['lin']['w']
    Wa, Wb, Wc = Wfull[:_HC], Wfull[_HC:2 * _HC], Wfull[2 * _HC:]
    bc = I['lin']['b'].reshape(1, _HC)
    W1 = I['lin_rbf_1']['w']

    x = _k_emb(z2d, emb)
    xi, xj = _sc_gather_xij(x, i, j)
    e1, e2 = _k_init(xi, xj, rbf, W0, b0, Wa, Wb, Wc, bc, W1)

    def update_v(V, e2v):
        vparts = _sc_segsum_v(e2v, i)
        return _k_out(
            vparts[:_N], vparts[_N:], batch2d,
            V['lin_up']['w'], V['lin_up']['b'].reshape(1, _OE),
            V['lins'][0]['w'], V['lins'][0]['b'].reshape(1, _OE),
            V['lins'][1]['w'], V['lins'][1]['b'].reshape(1, _OE),
            V['lins'][2]['w'], V['lins'][2]['b'].reshape(1, _OE),
            V['lin']['w'])

    u = update_v(P['update_v'][0], e2)

    for li in range(_NL):
        L = P['update_e'][li]
        Wr = jnp.dot(L['lin_rbf1']['w'], L['lin_rbf2']['w'])
        Ws = jnp.dot(L['lin_sbf1']['w'], L['lin_sbf2']['w'])
        xji, xd = _k_pre(
            e1, rbf,
            L['lin_ji']['w'], L['lin_ji']['b'].reshape(1, _HC),
            L['lin_kj']['w'], L['lin_kj']['b'].reshape(1, _HC),
            Wr, L['lin_down']['w'])
        s_tab = _k_s(sbf, Ws)
        agg = _sc_triplet(xd, s_tab, skj_p, sji_p, perm_p, bounds)
        e1, e2 = _k_post(
            agg, xji, e1, rbf,
            L['lin_up']['w'],
            L['before'][0]['lin1']['w'], L['before'][0]['lin1']['b'].reshape(1, _HC),
            L['before'][0]['lin2']['w'], L['before'][0]['lin2']['b'].reshape(1, _HC),
            L['lin']['w'], L['lin']['b'].reshape(1, _HC),
            L['after'][0]['lin1']['w'], L['after'][0]['lin1']['b'].reshape(1, _HC),
            L['after'][0]['lin2']['w'], L['after'][0]['lin2']['b'].reshape(1, _HC),
            L['after'][1]['lin1']['w'], L['after'][1]['lin1']['b'].reshape(1, _HC),
            L['after'][1]['lin2']['w'], L['after'][1]['lin2']['b'].reshape(1, _HC),
            L['lin_rbf']['w'])
        u = u + update_v(P['update_v'][li + 1], e2)

    return u


# speedup vs baseline: 6.3684x; 1.0017x over previous
import functools

import jax
import jax.numpy as jnp
from jax import lax
from jax.experimental import pallas as pl
from jax.experimental.pallas import tpu as pltpu
from jax.experimental.pallas import tpu_sc as plsc

_N = 10000
_E = 160000
_T = 640000
_NG = 64
_HC = 128
_IE = 64
_NR = 6
_SBF = 42
_OE = 256
_NL = 4

_NC = 2
_NS = 16

_BLK_E = 640
_BLK_T = 1024
_BLK_N = 400
_CHUNK_E = 10240
_NCHUNK = 16
_CT = 256
_CG = 200

_f32 = jnp.float32
_i32 = jnp.int32


def _swish(x):
    return x * (1.0 / (1.0 + jnp.exp(-x)))


def _full_spec(a):
    nd = a.ndim
    return pl.BlockSpec(a.shape, lambda b, _nd=nd: (0,) * _nd)


def _row_spec(blk, d):
    return pl.BlockSpec((blk, d), lambda b: (b, 0))


def _edense(body, grid, data_specs, weights, out_shapes, out_specs, *data):
    in_specs = list(data_specs) + [_full_spec(w) for w in weights]
    return pl.pallas_call(
        body,
        grid=grid,
        in_specs=in_specs,
        out_specs=out_specs,
        out_shape=out_shapes,
    )(*data, *weights)


def _k_emb(z2d, table):
    def body(z_ref, t_ref, o_ref):
        z = z_ref[...]
        oh = (z == lax.broadcasted_iota(_i32, (1, 128), 1)).astype(_f32)
        o_ref[...] = jnp.dot(oh, t_ref[...], preferred_element_type=_f32)

    return _edense(
        body, (_N // _BLK_N,), [_row_spec(_BLK_N, 1)], [table],
        jax.ShapeDtypeStruct((_N, _HC), _f32), _row_spec(_BLK_N, _HC), z2d)


def _k_init(xi, xj, rbf, W0, b0, Wa, Wb, Wc, bc, W1):
    def body(xi_ref, xj_ref, rbf_ref, W0r, b0r, War, Wbr, Wcr, bcr, W1r,
             e1_ref, e2_ref):
        rbf_v = rbf_ref[...]
        rbf0 = _swish(jnp.dot(rbf_v, W0r[...], preferred_element_type=_f32)
                      + b0r[...])
        e1 = _swish(jnp.dot(xi_ref[...], War[...], preferred_element_type=_f32)
                    + jnp.dot(xj_ref[...], Wbr[...], preferred_element_type=_f32)
                    + jnp.dot(rbf0, Wcr[...], preferred_element_type=_f32)
                    + bcr[...])
        e1_ref[...] = e1
        e2_ref[...] = jnp.dot(rbf_v, W1r[...], preferred_element_type=_f32) * e1

    return _edense(
        body, (_E // _BLK_E,),
        [_row_spec(_BLK_E, _HC), _row_spec(_BLK_E, _HC), _row_spec(_BLK_E, _NR)],
        [W0, b0, Wa, Wb, Wc, bc, W1],
        [jax.ShapeDtypeStruct((_E, _HC), _f32)] * 2,
        [_row_spec(_BLK_E, _HC)] * 2,
        xi, xj, rbf)


def _k_pre(e1, rbf, Wji, bji, Wkj, bkj, Wr, Wd):
    def body(x_ref, rbf_ref, Wjir, bjir, Wkjr, bkjr, Wrr, Wdr,
             xji_ref, xd_ref):
        x1 = x_ref[...]
        xji_ref[...] = _swish(
            jnp.dot(x1, Wjir[...], preferred_element_type=_f32) + bjir[...])
        xkj = _swish(
            jnp.dot(x1, Wkjr[...], preferred_element_type=_f32) + bkjr[...])
        r = jnp.dot(rbf_ref[...], Wrr[...], preferred_element_type=_f32)
        xd_ref[...] = _swish(
            jnp.dot(xkj * r, Wdr[...], preferred_element_type=_f32))

    return _edense(
        body, (_E // _BLK_E,),
        [_row_spec(_BLK_E, _HC), _row_spec(_BLK_E, _NR)],
        [Wji, bji, Wkj, bkj, Wr, Wd],
        [jax.ShapeDtypeStruct((_E, _HC), _f32),
         jax.ShapeDtypeStruct((_E, _IE), _f32)],
        [_row_spec(_BLK_E, _HC), _row_spec(_BLK_E, _IE)],
        e1, rbf)


def _k_s(sbf, Ws):
    def body(s_ref, Wr, o_ref):
        o_ref[...] = jnp.dot(s_ref[...], Wr[...], preferred_element_type=_f32)

    return _edense(
        body, (_T // _BLK_T,), [_row_spec(_BLK_T, _SBF)], [Ws],
        jax.ShapeDtypeStruct((_T, _IE), _f32), _row_spec(_BLK_T, _IE), sbf)


def _k_post(agg, xji, x1, rbf, Wup, Wb1, bb1, Wb2, bb2, Wl, bl,
            Wa11, ba11, Wa12, ba12, Wa21, ba21, Wa22, ba22, Wrbf):
    def body(agg_ref, xji_ref, x1_ref, rbf_ref, Wupr, Wb1r, bb1r, Wb2r, bb2r,
             Wlr, blr, Wa11r, ba11r, Wa12r, ba12r, Wa21r, ba21r, Wa22r, ba22r,
             Wrbfr, e1_ref, e2_ref):
        def linr(v, w, b):
            return jnp.dot(v, w[...], preferred_element_type=_f32) + b[...]

        xup = _swish(jnp.dot(agg_ref[...], Wupr[...],
                             preferred_element_type=_f32))
        e1 = xji_ref[...] + xup
        e1 = e1 + _swish(linr(_swish(linr(e1, Wb1r, bb1r)), Wb2r, bb2r))
        e1 = _swish(linr(e1, Wlr, blr)) + x1_ref[...]
        e1 = e1 + _swish(linr(_swish(linr(e1, Wa11r, ba11r)), Wa12r, ba12r))
        e1 = e1 + _swish(linr(_swish(linr(e1, Wa21r, ba21r)), Wa22r, ba22r))
        e1_ref[...] = e1
        e2_ref[...] = jnp.dot(rbf_ref[...], Wrbfr[...],
                              preferred_element_type=_f32) * e1

    return _edense(
        body, (_E // _BLK_E,),
        [_row_spec(_BLK_E, _IE), _row_spec(_BLK_E, _HC),
         _row_spec(_BLK_E, _HC), _row_spec(_BLK_E, _NR)],
        [Wup, Wb1, bb1, Wb2, bb2, Wl, bl, Wa11, ba11, Wa12, ba12,
         Wa21, ba21, Wa22, ba22, Wrbf],
        [jax.ShapeDtypeStruct((_E, _HC), _f32)] * 2,
        [_row_spec(_BLK_E, _HC)] * 2,
        agg, xji, x1, rbf)


def _k_out(va, vb, batch2d, Wup, bup, W1, b1, W2, b2, W3, b3, Wf):
    def body(va_ref, vb_ref, bt_ref, Wupr, bupr, W1r, b1r, W2r, b2r, W3r, b3r,
             Wfr, u_ref):
        v = va_ref[...] + vb_ref[...]
        v = jnp.dot(v, Wupr[...], preferred_element_type=_f32) + bupr[...]
        v = _swish(jnp.dot(v, W1r[...], preferred_element_type=_f32) + b1r[...])
        v = _swish(jnp.dot(v, W2r[...], preferred_element_type=_f32) + b2r[...])
        v = _swish(jnp.dot(v, W3r[...], preferred_element_type=_f32) + b3r[...])
        vo = jnp.dot(v, Wfr[...], preferred_element_type=_f32)
        oh = (bt_ref[...] == lax.broadcasted_iota(_i32, (1, _NG), 1)
              ).astype(_f32)
        u = lax.dot_general(oh, vo, (((0,), (0,)), ((), ())),
                            preferred_element_type=_f32)

        @pl.when(pl.program_id(0) == 0)
        def _():
            u_ref[...] = jnp.zeros_like(u_ref)

        u_ref[...] += u

    return _edense(
        body, (_N // _BLK_N,),
        [_row_spec(_BLK_N, _HC), _row_spec(_BLK_N, _HC), _row_spec(_BLK_N, 1)],
        [Wup, bup, W1, b1, W2, b2, W3, b3, Wf],
        jax.ShapeDtypeStruct((_NG, 1), _f32),
        pl.BlockSpec((_NG, 1), lambda b: (0, 0)),
        va, vb, batch2d)


def _sc_mesh():
    return plsc.VectorSubcoreMesh(core_axis_name="c", subcore_axis_name="s",
                                  num_cores=_NC, num_subcores=_NS)


def _zero_vmem(ref, rows, cols):
    z16 = jnp.zeros((16,), _f32)

    def body(r, carry):
        for cc in range(cols // 16):
            ref[r, pl.ds(cc * 16, 16)] = z16
        return carry

    lax.fori_loop(0, rows, body, None)


def _sc_gather_xij(x, iarr, jarr):
    per_tec = _E // (_NC * _NS)

    @functools.partial(
        pl.kernel,
        out_type=[jax.ShapeDtypeStruct((_E, _HC), _f32)] * 2,
        mesh=_sc_mesh(),
        compiler_params=pltpu.CompilerParams(needs_layout_passes=False, use_tc_tiling_on_sc=False),
        scratch_types=[
            pltpu.MemorySpace.VMEM((_CG,), _i32),
            pltpu.MemorySpace.VMEM((_CG, _HC), _f32),
            pltpu.SemaphoreType.DMA,
        ],
    )
    def k(x_hbm, i_hbm, j_hbm, xi_hbm, xj_hbm, idx_v, row_v, sem):
        c = lax.axis_index("c")
        s = lax.axis_index("s")
        base = (c * _NS + s) * per_tec

        def body(kk, carry):
            t = pl.multiple_of(base + kk * _CG, 8)
            pltpu.sync_copy(i_hbm.at[pl.ds(t, _CG)], idx_v)
            pltpu.async_copy(x_hbm.at[idx_v], row_v, sem).wait()
            pltpu.sync_copy(row_v, xi_hbm.at[pl.ds(t, _CG)])
            pltpu.sync_copy(j_hbm.at[pl.ds(t, _CG)], idx_v)
            pltpu.async_copy(x_hbm.at[idx_v], row_v, sem).wait()
            pltpu.sync_copy(row_v, xj_hbm.at[pl.ds(t, _CG)])
            return carry

        lax.fori_loop(0, per_tec // _CG, body, None)

    return k(x, iarr, jarr)


def _sc_segsum_v(e2, iarr):
    ncw = _E // _NC // 128 // _NS
    zrows = 80

    @functools.partial(
        pl.kernel,
        out_type=jax.ShapeDtypeStruct((_NC * _N, _HC), _f32),
        mesh=_sc_mesh(),
        compiler_params=pltpu.CompilerParams(needs_layout_passes=False, use_tc_tiling_on_sc=False),
        scratch_types=[
            pltpu.MemorySpace.VMEM_SHARED((_N, _HC), _f32),
            [pltpu.MemorySpace.VMEM((128,), _i32)] * 2,
            [pltpu.MemorySpace.VMEM((128, _HC), _f32)] * 2,
            pltpu.MemorySpace.VMEM((zrows, _HC), _f32),
            [pltpu.SemaphoreType.DMA] * 2,
            [pltpu.SemaphoreType.DMA] * 2,
        ],
    )
    def k(e2_hbm, i_hbm, out_hbm, acc, idx_v, row_v, zbuf, semr, semw):
        c = lax.axis_index("c")
        s = lax.axis_index("s")
        _zero_vmem(zbuf, zrows, _HC)
        nz = jnp.where(s == 15, 400 // zrows, 640 // zrows)

        def zb(q, carry):
            off = pl.multiple_of(s * 640 + q * zrows, 8)
            pltpu.sync_copy(zbuf, acc.at[pl.ds(off, zrows)])
            return carry

        lax.fori_loop(0, nz, zb, None)
        plsc.subcore_barrier()

        nch = _E // _NC // 128
        base_ch = c * nch

        def t_of(kk):
            return pl.multiple_of((base_ch + s + kk * _NS) * 128, 8)

        nk = (nch - s + _NS - 1) // _NS

        def wait_sc(b):
            pltpu.make_async_copy(row_v[b], acc.at[idx_v[b]], semw[b]).wait()

        def fire(kk, b):
            t = t_of(kk)
            pltpu.sync_copy(i_hbm.at[pl.ds(t, 128)], idx_v[b])
            pltpu.async_copy(e2_hbm.at[pl.ds(t, 128)], row_v[b], semr[b])

        @pl.when(nk > 0)
        def _():
            fire(0, 0)

        def body(kk, carry):
            def consume(b):
                @pl.when(kk + 1 < nk)
                def _():
                    @pl.when(kk + 1 >= 2)
                    def _():
                        wait_sc(1 - b)

                    fire(kk + 1, 1 - b)

                pltpu.make_async_copy(
                    e2_hbm.at[pl.ds(t_of(kk), 128)], row_v[b], semr[b]).wait()
                pltpu.async_copy(row_v[b], acc.at[idx_v[b]], semw[b],
                                 add=True)

            @pl.when(kk % 2 == 0)
            def _():
                consume(0)

            @pl.when(kk % 2 == 1)
            def _():
                consume(1)

            return carry

        lax.fori_loop(0, nk, body, None)
        last = nk - 1

        @pl.when(nk > 0)
        def _():
            @pl.when(last % 2 == 0)
            def _():
                wait_sc(0)

            @pl.when(last % 2 == 1)
            def _():
                wait_sc(1)

        @pl.when(nk > 1)
        def _():
            @pl.when(last % 2 == 0)
            def _():
                wait_sc(1)

            @pl.when(last % 2 == 1)
            def _():
                wait_sc(0)

        plsc.subcore_barrier()

        @pl.when(s != 15)
        def _():
            off = pl.multiple_of(s * 640, 8)
            pltpu.sync_copy(acc.at[pl.ds(off, 640)],
                            out_hbm.at[pl.ds(pl.multiple_of(
                                c * _N + off, 8), 640)])

        @pl.when(s == 15)
        def _():
            off = pl.multiple_of(15 * 640, 8)
            pltpu.sync_copy(acc.at[pl.ds(off, 400)],
                            out_hbm.at[pl.ds(pl.multiple_of(
                                c * _N + off, 8), 400)])

    return k(e2, iarr)


def _sc_triplet(xd, s_tab, skj_p, sji_p, perm_p, bounds):
    @functools.partial(
        pl.kernel,
        out_type=jax.ShapeDtypeStruct((_E, _IE), _f32),
        mesh=_sc_mesh(),
        compiler_params=pltpu.CompilerParams(needs_layout_passes=False, use_tc_tiling_on_sc=False),
        scratch_types=[
            pltpu.MemorySpace.VMEM_SHARED((_CHUNK_E + 16, _IE), _f32),
            [pltpu.MemorySpace.VMEM((_CT,), _i32)] * 2,
            [pltpu.MemorySpace.VMEM((_CT,), _i32)] * 2,
            [pltpu.MemorySpace.VMEM((_CT,), _i32)] * 2,
            [pltpu.MemorySpace.VMEM((_CT,), _i32)] * 2,
            pltpu.MemorySpace.VMEM((32,), _i32),
            [pltpu.MemorySpace.VMEM((_CT, _IE), _f32)] * 2,
            [pltpu.MemorySpace.VMEM((_CT, _IE), _f32)] * 2,
            [pltpu.SemaphoreType.DMA] * 2,
            [pltpu.SemaphoreType.DMA] * 2,
            [pltpu.SemaphoreType.DMA] * 2,
        ],
    )
    def k(xd_hbm, s_hbm, kj_hbm, ji_hbm, pm_hbm, bnd_hbm, agg_hbm,
          acc, kj_v, pm_v, ji_v, li_v, bnd_v, xrow_v, srow_v,
          semx, sems, semw):
        c = lax.axis_index("c")
        s = lax.axis_index("s")
        pltpu.sync_copy(bnd_hbm, bnd_v)
        iota16 = lax.iota(_i32, 16)

        def _bnd(ix):
            q = pl.multiple_of(ix // 16 * 16, 16)
            b16 = bnd_v[pl.ds(q, 16)]
            return jnp.sum(jnp.where(iota16 == ix % 16, b16, 0))

        def chunk_body(kk, carry):
            ch = c + _NC * kk
            base_e = ch * _CHUNK_E
            rows_c = jnp.minimum(_CHUNK_E, _E - base_e)
            t0 = _bnd(ch)
            t1 = _bnd(ch + 1)
            _zero_vmem(xrow_v[0], _CT, _IE)
            z0 = pl.multiple_of(s * 640, 8)
            pltpu.sync_copy(xrow_v[0], acc.at[pl.ds(z0, _CT)])
            pltpu.sync_copy(xrow_v[0], acc.at[pl.ds(
                pl.multiple_of(z0 + _CT, 8), _CT)])
            pltpu.sync_copy(xrow_v[0].at[pl.ds(0, 128)],
                            acc.at[pl.ds(pl.multiple_of(z0 + 2 * _CT, 8), 128)])

            @pl.when(s == 0)
            def _():
                pltpu.sync_copy(xrow_v[0].at[pl.ds(0, 16)],
                                acc.at[pl.ds(_CHUNK_E, 16)])

            plsc.subcore_barrier()
            span = t1 - t0
            st = t0 + span * s // _NS
            en = t0 + span * (s + 1) // _NS
            sal = (st // 16) * 16
            nsub = (en - sal + _CT - 1) // _CT

            def wait_sc(b):
                pltpu.make_async_copy(
                    xrow_v[b], acc.at[li_v[b]], semw[b]).wait()

            def fire(m, b):
                @pl.when(m >= 2)
                def _():
                    wait_sc(b)

                t = pl.multiple_of(sal + m * _CT, 8)
                pltpu.sync_copy(kj_hbm.at[pl.ds(t, _CT)], kj_v[b])
                pltpu.sync_copy(pm_hbm.at[pl.ds(t, _CT)], pm_v[b])
                pltpu.sync_copy(ji_hbm.at[pl.ds(t, _CT)], ji_v[b])
                pltpu.async_copy(xd_hbm.at[kj_v[b]], xrow_v[b], semx[b])
                pltpu.async_copy(s_hbm.at[pm_v[b]], srow_v[b], sems[b])

            @pl.when(nsub > 0)
            def _():
                fire(0, 0)

            def consume(m, b):
                t = pl.multiple_of(sal + m * _CT, 8)

                @pl.when(m + 1 < nsub)
                def _():
                    fire(m + 1, 1 - b)

                pltpu.make_async_copy(
                    xd_hbm.at[kj_v[b]], xrow_v[b], semx[b]).wait()
                pltpu.make_async_copy(
                    s_hbm.at[pm_v[b]], srow_v[b], sems[b]).wait()

                def liq(q, carry3):
                    pos = t + q * 16 + iota16
                    ji16 = ji_v[b][pl.ds(q * 16, 16)]
                    ok = (pos >= st) & (pos < en)
                    li_v[b][pl.ds(q * 16, 16)] = jnp.where(
                        ok, ji16 - base_e, _CHUNK_E)
                    return carry3

                lax.fori_loop(0, _CT // 16, liq, None)

                def mulq(r, carry3):
                    for rr in range(4):
                        for cc in range(_IE // 16):
                            sl = pl.ds(cc * 16, 16)
                            row = r * 4 + rr
                            xrow_v[b][row, sl] = (xrow_v[b][row, sl]
                                                  * srow_v[b][row, sl])
                    return carry3

                lax.fori_loop(0, _CT // 4, mulq, None)
                pltpu.async_copy(xrow_v[b], acc.at[li_v[b]], semw[b],
                                 add=True)

            def sub_body(m, carry2):
                @pl.when(m % 2 == 0)
                def _():
                    consume(m, 0)

                @pl.when(m % 2 == 1)
                def _():
                    consume(m, 1)

                return carry2

            lax.fori_loop(0, nsub, sub_body, None)
            last = nsub - 1

            @pl.when(nsub > 0)
            def _():
                @pl.when(last % 2 == 0)
                def _():
                    wait_sc(0)

                @pl.when(last % 2 == 1)
                def _():
                    wait_sc(1)

            @pl.when(nsub > 1)
            def _():
                @pl.when(last % 2 == 0)
                def _():
                    wait_sc(1)

                @pl.when(last % 2 == 1)
                def _():
                    wait_sc(0)

            plsc.subcore_barrier()
            full = _CHUNK_E // _NS
            tail = (_E - (_NCHUNK - 1) * _CHUNK_E) // _NS

            @pl.when(rows_c == _CHUNK_E)
            def _():
                off = pl.multiple_of(s * full, 8)
                pltpu.sync_copy(acc.at[pl.ds(off, full)],
                                agg_hbm.at[pl.ds(pl.multiple_of(
                                    base_e + off, 8), full)])

            @pl.when(rows_c != _CHUNK_E)
            def _():
                off = pl.multiple_of(s * tail, 8)
                pltpu.sync_copy(acc.at[pl.ds(off, tail)],
                                agg_hbm.at[pl.ds(pl.multiple_of(
                                    base_e + off, 8), tail)])

            plsc.subcore_barrier()
            return carry

        lax.fori_loop(0, _NCHUNK // _NC, chunk_body, None)

    return k(xd, s_tab, skj_p, sji_p, perm_p, bounds)


def kernel(z, rbf, sbf, i, j, idx_kj, idx_ji, batch, params):
    z = z.astype(_i32)
    i = i.astype(_i32)
    j = j.astype(_i32)
    idx_kj = idx_kj.astype(_i32)
    idx_ji = idx_ji.astype(_i32)
    batch = batch.astype(_i32)

    perm = jnp.argsort(idx_ji).astype(_i32)
    sji = jnp.take(idx_ji, perm)
    skj = jnp.take(idx_kj, perm)
    chunk_edges = jnp.arange(_NCHUNK + 1, dtype=_i32) * _CHUNK_E
    bounds = jnp.searchsorted(sji, chunk_edges, side='left').astype(_i32)
    bounds = jnp.concatenate(
        [bounds, jnp.zeros((32 - _NCHUNK - 1,), _i32)])
    padi = jnp.zeros((_CT,), _i32)
    sji_p = jnp.concatenate([sji, jnp.full((_CT,), _E, _i32)])
    skj_p = jnp.concatenate([skj, padi])
    perm_p = jnp.concatenate([perm, padi])

    z2d = z.reshape(_N, 1)
    batch2d = batch.reshape(_N, 1)

    P = params
    emb = jnp.zeros((128, _HC), _f32).at[:95].set(P['emb_table'])
    I = P['init']
    W0 = I['lin_rbf_0']['w']
    b0 = I['lin_rbf_0']['b'].reshape(1, _HC)
    Wfull = I['lin']['w']
    Wa, Wb, Wc = Wfull[:_HC], Wfull[_HC:2 * _HC], Wfull[2 * _HC:]
    bc = I['lin']['b'].reshape(1, _HC)
    W1 = I['lin_rbf_1']['w']

    x = _k_emb(z2d, emb)
    xi, xj = _sc_gather_xij(x, i, j)
    e1, e2 = _k_init(xi, xj, rbf, W0, b0, Wa, Wb, Wc, bc, W1)

    def update_v(V, e2v):
        vparts = _sc_segsum_v(e2v, i)
        return _k_out(
            vparts[:_N], vparts[_N:], batch2d,
            V['lin_up']['w'], V['lin_up']['b'].reshape(1, _OE),
            V['lins'][0]['w'], V['lins'][0]['b'].reshape(1, _OE),
            V['lins'][1]['w'], V['lins'][1]['b'].reshape(1, _OE),
            V['lins'][2]['w'], V['lins'][2]['b'].reshape(1, _OE),
            V['lin']['w'])

    u = update_v(P['update_v'][0], e2)

    for li in range(_NL):
        L = P['update_e'][li]
        Wr = jnp.dot(L['lin_rbf1']['w'], L['lin_rbf2']['w'])
        Ws = jnp.dot(L['lin_sbf1']['w'], L['lin_sbf2']['w'])
        xji, xd = _k_pre(
            e1, rbf,
            L['lin_ji']['w'], L['lin_ji']['b'].reshape(1, _HC),
            L['lin_kj']['w'], L['lin_kj']['b'].reshape(1, _HC),
            Wr, L['lin_down']['w'])
        s_tab = _k_s(sbf, Ws)
        agg = _sc_triplet(xd, s_tab, skj_p, sji_p, perm_p, bounds)
        e1, e2 = _k_post(
            agg, xji, e1, rbf,
            L['lin_up']['w'],
            L['before'][0]['lin1']['w'], L['before'][0]['lin1']['b'].reshape(1, _HC),
            L['before'][0]['lin2']['w'], L['before'][0]['lin2']['b'].reshape(1, _HC),
            L['lin']['w'], L['lin']['b'].reshape(1, _HC),
            L['after'][0]['lin1']['w'], L['after'][0]['lin1']['b'].reshape(1, _HC),
            L['after'][0]['lin2']['w'], L['after'][0]['lin2']['b'].reshape(1, _HC),
            L['after'][1]['lin1']['w'], L['after'][1]['lin1']['b'].reshape(1, _HC),
            L['after'][1]['lin2']['w'], L['after'][1]['lin2']['b'].reshape(1, _HC),
            L['lin_rbf']['w'])
        u = u + update_v(P['update_v'][li + 1], e2)

    return u
